# Initial kernel scaffold; baseline (speedup 1.0000x reference)
import jax
import jax.numpy as jnp
from jax.experimental import pallas as pl


def kernel(x, edge_index, batch, Wl0, bl0, Wr0, g0, be0, Wl1, bl1, Wr1, g1, be1, Wl2, bl2, Wr2, g2, be2, fc1_W, fc1_b, fc2_W, fc2_b):
    raise NotImplementedError("write your pallas kernel here")


# speedup vs baseline: 4.6666x; 4.6666x over previous
import functools

import jax
import jax.numpy as jnp
from jax import lax
from jax.experimental import pallas as pl
from jax.experimental.pallas import tpu as pltpu
from jax.experimental.pallas import tpu_sc as plsc

N = 10000
E = 320000
D = 128
G = 64
NC = 2
NS = 16
EPW = E // (NC * NS)
CH = 80
NIT = EPW // CH
RPS = 624
CW = 16


def _mm_t(a, b):
    return lax.dot_general(a, b, (((1,), (1,)), ((), ())),
                           preferred_element_type=jnp.float32)


def _prep_body(x_ref, wl_ref, wr_ref, u_ref, v_ref):
    xv = x_ref[...]
    u_ref[...] = _mm_t(xv, wl_ref[...])
    v_ref[...] = _mm_t(xv, wr_ref[...])


_tc_prep = pl.pallas_call(
    _prep_body,
    out_shape=(jax.ShapeDtypeStruct((N, D), jnp.float32),
               jax.ShapeDtypeStruct((N, D), jnp.float32)),
)


def _bn_relu(s0, s1, c0, c1, v, bl, g, be):
    cnt = jnp.maximum(c0[:, 0:1] + c1[:, 0:1], 1.0)
    h = (s0 + s1) / cnt + bl + v
    m = jnp.mean(h, axis=0, keepdims=True)
    var = jnp.mean(jnp.square(h - m), axis=0, keepdims=True)
    h = g * (h - m) * lax.rsqrt(var + 1e-5) + be
    return jnp.maximum(h, 0.0)


def _norm_body(s0_ref, s1_ref, c0_ref, c1_ref, v_ref, bl_ref, g_ref, be_ref,
               wl_ref, wr_ref, u_ref, vo_ref):
    h = _bn_relu(s0_ref[...], s1_ref[...], c0_ref[...], c1_ref[...],
                 v_ref[...], bl_ref[...], g_ref[...], be_ref[...])
    u_ref[...] = _mm_t(h, wl_ref[...])
    vo_ref[...] = _mm_t(h, wr_ref[...])


_tc_norm = pl.pallas_call(
    _norm_body,
    out_shape=(jax.ShapeDtypeStruct((N, D), jnp.float32),
               jax.ShapeDtypeStruct((N, D), jnp.float32)),
)


def _final_body(s0_ref, s1_ref, c0_ref, c1_ref, v_ref, bl_ref, g_ref, be_ref,
                batch_ref, f1w_ref, f1b_ref, f2w_ref, f2b_ref, out_ref):
    h = _bn_relu(s0_ref[...], s1_ref[...], c0_ref[...], c1_ref[...],
                 v_ref[...], bl_ref[...], g_ref[...], be_ref[...])
    oh = (batch_ref[...] == lax.broadcasted_iota(jnp.int32, (N, G), 1)
          ).astype(jnp.float32)
    sums = lax.dot_general(oh, h, (((0,), (0,)), ((), ())),
                           preferred_element_type=jnp.float32)
    cnts = jnp.sum(oh, axis=0)[:, None]
    pooled = sums / jnp.maximum(cnts, 1.0)
    z = jnp.maximum(_mm_t(pooled, f1w_ref[...]) + f1b_ref[...], 0.0)
    out_ref[...] = _mm_t(z, f2w_ref[...]) + f2b_ref[...]


_tc_final = pl.pallas_call(
    _final_body,
    out_shape=jax.ShapeDtypeStruct((G, 10), jnp.float32),
)


def _make_sc_agg(with_cnt):
    mesh = plsc.VectorSubcoreMesh(core_axis_name="c", subcore_axis_name="s")
    outs = [jax.ShapeDtypeStruct((N, D), jnp.float32)] * 2
    scratch = [
        pltpu.VMEM((CH,), jnp.int32),
        pltpu.VMEM((CH,), jnp.int32),
        pltpu.VMEM((CH, D), jnp.float32),
        pltpu.VMEM_SHARED((N, D), jnp.float32),
        pltpu.SemaphoreType.DMA,
    ]
    if with_cnt:
        outs = outs + [jax.ShapeDtypeStruct((N,), jnp.float32)] * 2
        scratch = scratch + [
            pltpu.VMEM((16,), jnp.float32),
            pltpu.VMEM_SHARED((N,), jnp.float32),
            pltpu.VMEM((RPS,), jnp.float32),
        ]

    def body(u, src_a, dst_a, zrows, *rest):
        if with_cnt:
            (zcnt, s0, s1, c0, c1,
             idx_s, idx_d, rows, acc, sem, onesv, cacc, bounce) = rest
        else:
            (s0, s1, idx_s, idx_d, rows, acc, sem) = rest
        cid = lax.axis_index("c")
        sid = lax.axis_index("s")
        row0 = sid * RPS
        tail0 = NS * RPS

        def stripe_copy(src_ref, dst_ref):
            pltpu.sync_copy(src_ref.at[pl.ds(row0, RPS)],
                            dst_ref.at[pl.ds(row0, RPS)])

            @pl.when(sid == NS - 1)
            def _():
                pltpu.sync_copy(src_ref.at[pl.ds(tail0, N - tail0)],
                                dst_ref.at[pl.ds(tail0, N - tail0)])

        def stripe_copy_1d(src_ref, dst_ref):
            pltpu.sync_copy(src_ref.at[pl.ds(row0, RPS)], bounce)
            pltpu.sync_copy(bounce, dst_ref.at[pl.ds(row0, RPS)])

            @pl.when(sid == NS - 1)
            def _():
                t = N - tail0
                pltpu.sync_copy(src_ref.at[pl.ds(tail0, t)],
                                bounce.at[pl.ds(0, t)])
                pltpu.sync_copy(bounce.at[pl.ds(0, t)],
                                dst_ref.at[pl.ds(tail0, t)])

        stripe_copy(zrows, acc)
        if with_cnt:
            stripe_copy_1d(zcnt, cacc)
            onesv[...] = jnp.full((16,), 1.0, jnp.float32)
        plsc.subcore_barrier()

        base = (cid * NS + sid) * EPW

        def step(i, carry):
            off = base + i * CH
            pltpu.sync_copy(src_a.at[pl.ds(off, CH)], idx_s)
            pltpu.sync_copy(dst_a.at[pl.ds(off, CH)], idx_d)
            pltpu.async_copy(u.at[idx_s], rows, sem).wait()
            pltpu.sync_copy(rows, acc.at[idx_d], add=True)
            if with_cnt:
                for j in range(CH // 16):
                    dv = idx_d[pl.ds(j * 16, 16)]
                    pltpu.sync_copy(onesv, cacc.at[dv], add=True)
            return carry

        lax.fori_loop(0, NIT, step, 0)
        plsc.subcore_barrier()

        @pl.when(cid == 0)
        def _():
            stripe_copy(acc, s0)

        @pl.when(cid == 1)
        def _():
            stripe_copy(acc, s1)

        if with_cnt:
            @pl.when(cid == 0)
            def _():
                stripe_copy_1d(cacc, c0)

            @pl.when(cid == 1)
            def _():
                stripe_copy_1d(cacc, c1)

    return pl.kernel(body, out_type=tuple(outs), mesh=mesh,
                     scratch_types=tuple(scratch))


_sc_agg_cnt = _make_sc_agg(True)
_sc_agg = _make_sc_agg(False)


def kernel(x, edge_index, batch,
           Wl0, bl0, Wr0, g0, be0,
           Wl1, bl1, Wr1, g1, be1,
           Wl2, bl2, Wr2, g2, be2,
           fc1_W, fc1_b, fc2_W, fc2_b):
    src = edge_index[0]
    dst = edge_index[1]
    zrows = jnp.zeros((N, D), jnp.float32)
    zcnt = jnp.zeros((N,), jnp.float32)
    r = lambda w: w.reshape(1, -1)

    u, v = _tc_prep(x, Wl0, Wr0)
    s0, s1, c0, c1 = _sc_agg_cnt(u, src, dst, zrows, zcnt)
    c0 = c0.reshape(N, 1)
    c1 = c1.reshape(N, 1)
    u, v = _tc_norm(s0, s1, c0, c1, v, r(bl0), r(g0), r(be0), Wl1, Wr1)
    s0, s1 = _sc_agg(u, src, dst, zrows)
    u, v = _tc_norm(s0, s1, c0, c1, v, r(bl1), r(g1), r(be1), Wl2, Wr2)
    s0, s1 = _sc_agg(u, src, dst, zrows)
    out = _tc_final(s0, s1, c0, c1, v, r(bl2), r(g2), r(be2),
                    batch.reshape(-1, 1), fc1_W, fc1_b, fc2_W, fc2_b)
    return out
